# baseline (speedup 1.0000x reference)
import functools

import jax
import jax.numpy as jnp
from jax import lax
from jax.experimental import pallas as pl
from jax.experimental.pallas import tpu as pltpu
from jax.experimental.pallas import tpu_sc as plsc

N = 10000
E = 320000
D_NODE = 128
D_EDGE = 16
G = 16
OUT = 128

BN = 2000
BEDGE = 2000


def _ln(h, g, b):
    mu = jnp.mean(h, axis=-1, keepdims=True)
    var = jnp.mean((h - mu) * (h - mu), axis=-1, keepdims=True)
    return (h - mu) * lax.rsqrt(var + 1e-5) * g + b


def _prep_body(x_ref, oh_ref, u_ref, wr_ref, ws_ref, wg_ref, b1_ref,
               ab_ref):
    xb = x_ref[...]
    ab_ref[0] = jnp.dot(xb, wr_ref[...], preferred_element_type=jnp.float32)
    ug = jnp.dot(u_ref[...], wg_ref[...], preferred_element_type=jnp.float32)
    ab_ref[1] = (jnp.dot(xb, ws_ref[...], preferred_element_type=jnp.float32)
                 + jnp.dot(oh_ref[...], ug, preferred_element_type=jnp.float32)
                 + b1_ref[...])


def _prep_tables(x, onehot, u, wr, ws, wg, b1):
    grid = N // BN
    full = lambda shape: pl.BlockSpec(shape, lambda i: (0, 0))
    return pl.pallas_call(
        _prep_body,
        grid=(grid,),
        in_specs=[
            pl.BlockSpec((BN, D_NODE), lambda i: (i, 0)),
            pl.BlockSpec((BN, G), lambda i: (i, 0)),
            full((G, D_NODE)),
            full((D_NODE, OUT)),
            full((D_NODE, OUT)),
            full((D_NODE, OUT)),
            full((1, OUT)),
        ],
        out_specs=pl.BlockSpec((2, BN, OUT), lambda i: (0, i, 0)),
        out_shape=jax.ShapeDtypeStruct((2, N, OUT), jnp.float32),
        compiler_params=pltpu.CompilerParams(
            dimension_semantics=("arbitrary",)),
    )(x, onehot, u, wr, ws, wg, b1)


def _edge_body(ga_ref, gb_ref, row_ref, st_ref, en_ref, e_ref, w1e_ref,
               w2_ref, b2_ref, g_ref, bt_ref, out_ref, ea_ref, acc_ref):
    i = pl.program_id(0)

    @pl.when(i == 0)
    def _():
        acc_ref[...] = jnp.zeros_like(acc_ref)

    h1 = jnp.maximum(
        ga_ref[...] + gb_ref[...]
        + jnp.dot(e_ref[...], w1e_ref[...],
                  preferred_element_type=jnp.float32), 0.0)
    h2 = jnp.maximum(
        jnp.dot(h1, w2_ref[...], preferred_element_type=jnp.float32)
        + b2_ref[...], 0.0)
    en = _ln(h2, g_ref[...], bt_ref[...])
    out_ref[...] = en
    rr = row_ref[0]
    ohgT = jnp.logical_and(rr >= st_ref[...], rr < en_ref[...]
                           ).astype(jnp.float32)
    acc_ref[...] += lax.dot_general(
        ohgT, en, (((1,), (0,)), ((), ())),
        preferred_element_type=jnp.float32)

    @pl.when(i == pl.num_programs(0) - 1)
    def _():
        ea_ref[...] = acc_ref[...]


def _edge_mlp(ga, gb, row3, starts, ends, e, w1e, w2, b2, g, bt,
              blk_off, n_blk):
    full = lambda shape: pl.BlockSpec(shape, lambda i: (0, 0))
    return pl.pallas_call(
        _edge_body,
        grid=(n_blk,),
        in_specs=[
            pl.BlockSpec((BEDGE, OUT), lambda i: (i, 0)),
            pl.BlockSpec((BEDGE, OUT), lambda i: (i, 0)),
            pl.BlockSpec((1, 1, BEDGE), lambda i: (i + blk_off, 0, 0)),
            pl.BlockSpec((G, 1), lambda i: (0, 0)),
            pl.BlockSpec((G, 1), lambda i: (0, 0)),
            pl.BlockSpec((BEDGE, D_EDGE), lambda i: (i + blk_off, 0)),
            full((D_EDGE, OUT)),
            full((OUT, OUT)),
            full((1, OUT)),
            full((1, OUT)),
            full((1, OUT)),
        ],
        out_specs=[
            pl.BlockSpec((BEDGE, OUT), lambda i: (i, 0)),
            pl.BlockSpec((G, OUT), lambda i: (0, 0)),
        ],
        out_shape=[
            jax.ShapeDtypeStruct((n_blk * BEDGE, OUT), jnp.float32),
            jax.ShapeDtypeStruct((G, OUT), jnp.float32),
        ],
        scratch_shapes=[pltpu.VMEM((G, OUT), jnp.float32)],
        compiler_params=pltpu.CompilerParams(
            dimension_semantics=("arbitrary",)),
    )(ga, gb, row3, starts, ends, e, w1e, w2, b2, g, bt)


def _node_body(x_ref, recv_ref, ea_ref, oh_ref, u_ref,
               nw1x_ref, nw1r_ref, nw1u_ref, nb1_ref, nw2_ref, nb2_ref,
               ng_ref, nbt_ref,
               gw1u_ref, gw1n_ref, gw1e_ref, gb1_ref, gw2_ref, gb2_ref,
               gg_ref, gbt_ref,
               xn_ref, un_ref, acc_ref):
    i = pl.program_id(0)

    @pl.when(i == 0)
    def _():
        acc_ref[...] = jnp.zeros_like(acc_ref)

    oh = oh_ref[...]
    un_tab = jnp.dot(u_ref[...], nw1u_ref[...],
                     preferred_element_type=jnp.float32)
    h = jnp.maximum(
        jnp.dot(x_ref[...], nw1x_ref[...], preferred_element_type=jnp.float32)
        + jnp.dot(recv_ref[...], nw1r_ref[...],
                  preferred_element_type=jnp.float32)
        + jnp.dot(oh, un_tab, preferred_element_type=jnp.float32)
        + nb1_ref[...], 0.0)
    h2 = jnp.maximum(
        jnp.dot(h, nw2_ref[...], preferred_element_type=jnp.float32)
        + nb2_ref[...], 0.0)
    xn = _ln(h2, ng_ref[...], nbt_ref[...])
    xn_ref[...] = xn

    contract0 = (((0,), (0,)), ((), ()))
    acc_ref[...] += lax.dot_general(
        oh, xn, contract0, preferred_element_type=jnp.float32)

    @pl.when(i == pl.num_programs(0) - 1)
    def _():
        na = acc_ref[...]
        ea = ea_ref[...]
        hg = jnp.maximum(
            jnp.dot(u_ref[...], gw1u_ref[...],
                    preferred_element_type=jnp.float32)
            + jnp.dot(na, gw1n_ref[...], preferred_element_type=jnp.float32)
            + jnp.dot(ea, gw1e_ref[...], preferred_element_type=jnp.float32)
            + gb1_ref[...], 0.0)
        hg2 = jnp.maximum(
            jnp.dot(hg, gw2_ref[...], preferred_element_type=jnp.float32)
            + gb2_ref[...], 0.0)
        un_ref[...] = _ln(hg2, gg_ref[...], gbt_ref[...])


def _node_global(x, recv, ea, onehot, u,
                 nw1x, nw1r, nw1u, nb1, nw2, nb2, ng, nbt,
                 gw1u, gw1n, gw1e, gb1, gw2, gb2, gg, gbt):
    grid = N // BN
    full = lambda shape: pl.BlockSpec(shape, lambda i: (0, 0))
    blk = lambda w: pl.BlockSpec((BN, w), lambda i: (i, 0))
    return pl.pallas_call(
        _node_body,
        grid=(grid,),
        in_specs=[
            blk(D_NODE), blk(OUT),
            full((G, OUT)),
            blk(G),
            full((G, D_NODE)),
            full((D_NODE, OUT)), full((OUT, OUT)), full((D_NODE, OUT)),
            full((1, OUT)), full((OUT, OUT)), full((1, OUT)),
            full((1, OUT)), full((1, OUT)),
            full((D_NODE, OUT)), full((OUT, OUT)), full((OUT, OUT)),
            full((1, OUT)), full((OUT, OUT)), full((1, OUT)),
            full((1, OUT)), full((1, OUT)),
        ],
        out_specs=[
            pl.BlockSpec((BN, OUT), lambda i: (i, 0)),
            pl.BlockSpec((G, OUT), lambda i: (0, 0)),
        ],
        out_shape=[
            jax.ShapeDtypeStruct((N, OUT), jnp.float32),
            jax.ShapeDtypeStruct((G, OUT), jnp.float32),
        ],
        scratch_shapes=[pltpu.VMEM((G, OUT), jnp.float32)],
        compiler_params=pltpu.CompilerParams(
            dimension_semantics=("arbitrary",)),
    )(x, recv, ea, onehot, u,
      nw1x, nw1r, nw1u, nb1, nw2, nb2, ng, nbt,
      gw1u, gw1n, gw1e, gb1, gw2, gb2, gg, gbt)


_SC_KB = 80
_SC_SUB = 8
_SC_CHUNK = _SC_KB * _SC_SUB

_NPAD = 10240
_HN = _NPAD // 2
_TRASH = _HN
_ACC = _HN + 8


def _sc_scatter2(e3_parts, ei4, zrows):
    info = plsc.get_sparse_core_info()
    ns = info.num_subcores
    part_rows = [p.shape[0] // _SC_SUB for p in e3_parts]
    stripe = _HN // ns
    mesh = plsc.VectorSubcoreMesh(core_axis_name="c", subcore_axis_name="s")

    @functools.partial(
        pl.kernel, mesh=mesh,
        out_type=jax.ShapeDtypeStruct((_NPAD, OUT), jnp.float32),
        scratch_types=[
            pltpu.VMEM((_SC_SUB, _SC_KB, OUT), jnp.float32),
            pltpu.VMEM((_SC_SUB, _SC_KB), jnp.int32),
            pltpu.VMEM_SHARED((_ACC, OUT), jnp.float32),
            pltpu.SemaphoreType.DMA,
        ],
    )
    def k(*refs):
        e_hbms = refs[:len(e3_parts)]
        ei_hbm, z_hbm, outr_hbm, data_v, idx_v, acc_sh, sem = \
            refs[len(e3_parts):]
        c = lax.axis_index("c")
        s = lax.axis_index("s")
        base = c * _HN

        pltpu.sync_copy(z_hbm.at[pl.ds(0, stripe)],
                        acc_sh.at[pl.ds(s * stripe, stripe)])
        plsc.subcore_barrier()

        def do_row(e_hbm, local, glob):
            cp1 = pltpu.make_async_copy(
                e_hbm.at[pl.ds(local * _SC_SUB, _SC_SUB)], data_v, sem)
            cp1.start()
            cp2 = pltpu.make_async_copy(ei_hbm.at[1, glob], idx_v, sem)
            cp2.start()
            cp1.wait()
            cp2.wait()
            for kk in range(_SC_SUB):
                for jj in range(_SC_KB // 16):
                    v = idx_v[kk, pl.ds(jj * 16, 16)] - base
                    oob = (v < 0) | (v >= _HN)
                    idx_v[kk, pl.ds(jj * 16, 16)] = jnp.where(oob, _TRASH, v)
            for kk in range(_SC_SUB):
                pltpu.sync_copy(data_v.at[kk], acc_sh.at[idx_v.at[kk]],
                                add=True)

        row_lo = 0
        for e_hbm, rows_p in zip(e_hbms, part_rows):
            n_rounds = rows_p // ns
            n_left = rows_p - n_rounds * ns

            def body(j, _, e_hbm=e_hbm, lo=row_lo):
                local = j * ns + s
                do_row(e_hbm, local, lo + local)
                return 0

            lax.fori_loop(0, n_rounds, body, 0)

            @pl.when(s < n_left)
            def _(e_hbm=e_hbm, lo=row_lo, n_rounds=n_rounds):
                local = n_rounds * ns + s
                do_row(e_hbm, local, lo + local)

            row_lo += rows_p

        plsc.subcore_barrier()
        pltpu.sync_copy(acc_sh.at[pl.ds(s * stripe, stripe)],
                        outr_hbm.at[pl.ds(base + s * stripe, stripe)])

    return k(*e3_parts, ei4, zrows)


def _sc_gather2(tab2, gi4, row_lo, row_n):
    info = plsc.get_sparse_core_info()
    ns = info.num_subcores
    n_rounds = row_n // ns
    n_left = row_n - n_rounds * ns
    mesh = plsc.VectorSubcoreMesh(core_axis_name="c", subcore_axis_name="s")

    @functools.partial(
        pl.kernel, mesh=mesh,
        out_type=jax.ShapeDtypeStruct((2, row_n * _SC_SUB, _SC_KB, OUT),
                                      jnp.float32),
        scratch_types=[
            pltpu.VMEM((_SC_SUB, _SC_KB, OUT), jnp.float32),
            pltpu.VMEM((_SC_SUB, _SC_KB), jnp.int32),
            pltpu.SemaphoreType.DMA,
        ],
    )
    def k(tab_hbm, gi_hbm, out_hbm, data_v, idx_v, sem):
        c = lax.axis_index("c")
        s = lax.axis_index("s")

        def do_row(local):
            pltpu.sync_copy(gi_hbm.at[c, row_lo + local], idx_v)
            for kk in range(_SC_SUB):
                pltpu.make_async_copy(
                    tab_hbm.at[idx_v.at[kk]], data_v.at[kk], sem).start()
            for kk in range(_SC_SUB):
                pltpu.make_async_copy(
                    tab_hbm.at[idx_v.at[kk]], data_v.at[kk], sem).wait()
            pltpu.sync_copy(
                data_v, out_hbm.at[c, pl.ds(local * _SC_SUB, _SC_SUB)])

        def body(j, _):
            do_row(j * ns + s)
            return 0

        lax.fori_loop(0, n_rounds, body, 0)

        @pl.when(s < n_left)
        def _():
            do_row(n_rounds * ns + s)

    return k(tab2, gi4)


def kernel(x, e, u, edge_index, batch,
           eW1, eb1, eW2, eb2, eg, ebt,
           nW1, nb1, nW2, nb2, ng, nbt,
           gW1, gb1, gW2, gb2, gg, gbt):
    row = edge_index[0]
    col = edge_index[1]
    onehot = (batch[:, None] == jnp.arange(G, dtype=jnp.int32)[None, :]
              ).astype(jnp.float32)

    r1 = lambda v: v.reshape(1, -1)

    w1e = eW1[:D_EDGE]
    w1r = eW1[D_EDGE:D_EDGE + D_NODE]
    w1s = eW1[D_EDGE + D_NODE:D_EDGE + 2 * D_NODE]
    w1g = eW1[D_EDGE + 2 * D_NODE:]

    ab_tab = _prep_tables(x, onehot, u, w1r, w1s, w1g, r1(eb1))
    tab2 = ab_tab.reshape(2 * N, OUT)
    gi4 = jnp.stack([col, row + N]).reshape(2, E // _SC_CHUNK,
                                            _SC_SUB, _SC_KB)
    cnt = jnp.sum(onehot, axis=0).astype(jnp.int32)
    ends_f = jnp.cumsum(cnt)
    starts = (ends_f - cnt).reshape(G, 1)
    ends = ends_f.reshape(G, 1)
    row3 = row.reshape(E // BEDGE, 1, BEDGE)

    n_rows_total = E // _SC_CHUNK
    half_rows = n_rows_total // 2
    half_blk = (half_rows * _SC_CHUNK) // BEDGE

    e_parts, ea_parts = [], []
    for p in range(2):
        g2 = _sc_gather2(tab2, gi4, p * half_rows, half_rows)
        ga = g2[0].reshape(half_rows * _SC_CHUNK, OUT)
        gb = g2[1].reshape(half_rows * _SC_CHUNK, OUT)
        en_p, ea_p = _edge_mlp(ga, gb, row3, starts, ends, e, w1e, eW2,
                               r1(eb2), r1(eg), r1(ebt),
                               p * half_blk, half_blk)
        e_parts.append(en_p)
        ea_parts.append(ea_p)

    ea = ea_parts[0] + ea_parts[1]
    e3_parts = [ep.reshape(half_rows * _SC_SUB, _SC_KB, OUT)
                for ep in e_parts]
    ei4 = edge_index.reshape(2, E // _SC_CHUNK, _SC_SUB, _SC_KB)
    zrows = jnp.zeros((320, OUT), jnp.float32)
    recv_full = _sc_scatter2(e3_parts, ei4, zrows)
    recv = recv_full[:N]
    e_new = jnp.concatenate(e_parts, axis=0)

    x_new, u_new = _node_global(
        x, recv, ea, onehot, u,
        nW1[:D_NODE], nW1[D_NODE:D_NODE + OUT], nW1[D_NODE + OUT:],
        r1(nb1), nW2, r1(nb2), r1(ng), r1(nbt),
        gW1[:D_NODE], gW1[D_NODE:D_NODE + OUT], gW1[D_NODE + OUT:],
        r1(gb1), gW2, r1(gb2), r1(gg), r1(gbt))

    return (x_new, e_new, u_new)


# speedup vs baseline: 1.0897x; 1.0897x over previous
import functools

import jax
import jax.numpy as jnp
from jax import lax
from jax.experimental import pallas as pl
from jax.experimental.pallas import tpu as pltpu
from jax.experimental.pallas import tpu_sc as plsc

N = 10000
E = 320000
D_NODE = 128
D_EDGE = 16
G = 16
OUT = 128

BN = 2000
BEDGE = 2000


def _ln(h, g, b):
    mu = jnp.mean(h, axis=-1, keepdims=True)
    var = jnp.mean((h - mu) * (h - mu), axis=-1, keepdims=True)
    return (h - mu) * lax.rsqrt(var + 1e-5) * g + b


def _prep_body(x_ref, oh_ref, u_ref, wr_ref, ws_ref, wg_ref, b1_ref,
               ab_ref):
    xb = x_ref[...]
    ab_ref[0] = jnp.dot(xb, wr_ref[...], preferred_element_type=jnp.float32)
    ug = jnp.dot(u_ref[...], wg_ref[...], preferred_element_type=jnp.float32)
    ab_ref[1] = (jnp.dot(xb, ws_ref[...], preferred_element_type=jnp.float32)
                 + jnp.dot(oh_ref[...], ug, preferred_element_type=jnp.float32)
                 + b1_ref[...])


def _prep_tables(x, onehot, u, wr, ws, wg, b1):
    grid = N // BN
    full = lambda shape: pl.BlockSpec(shape, lambda i: (0, 0))
    return pl.pallas_call(
        _prep_body,
        grid=(grid,),
        in_specs=[
            pl.BlockSpec((BN, D_NODE), lambda i: (i, 0)),
            pl.BlockSpec((BN, G), lambda i: (i, 0)),
            full((G, D_NODE)),
            full((D_NODE, OUT)),
            full((D_NODE, OUT)),
            full((D_NODE, OUT)),
            full((1, OUT)),
        ],
        out_specs=pl.BlockSpec((2, BN, OUT), lambda i: (0, i, 0)),
        out_shape=jax.ShapeDtypeStruct((2, N, OUT), jnp.float32),
        compiler_params=pltpu.CompilerParams(
            dimension_semantics=("arbitrary",)),
    )(x, onehot, u, wr, ws, wg, b1)


def _edge_body(ga_ref, gb_ref, row_ref, st_ref, en_ref, e_ref, w1e_ref,
               w2_ref, b2_ref, g_ref, bt_ref, out_ref, ea_ref, acc_ref):
    i = pl.program_id(0)

    @pl.when(i == 0)
    def _():
        acc_ref[...] = jnp.zeros_like(acc_ref)

    h1 = jnp.maximum(
        ga_ref[...] + gb_ref[...]
        + jnp.dot(e_ref[...], w1e_ref[...],
                  preferred_element_type=jnp.float32), 0.0)
    h2 = jnp.maximum(
        jnp.dot(h1, w2_ref[...], preferred_element_type=jnp.float32)
        + b2_ref[...], 0.0)
    en = _ln(h2, g_ref[...], bt_ref[...])
    out_ref[...] = en
    rr = row_ref[0]
    ohgT = jnp.logical_and(rr >= st_ref[...], rr < en_ref[...]
                           ).astype(jnp.float32)
    acc_ref[...] += lax.dot_general(
        ohgT, en, (((1,), (0,)), ((), ())),
        preferred_element_type=jnp.float32)

    @pl.when(i == pl.num_programs(0) - 1)
    def _():
        ea_ref[...] = acc_ref[...]


def _edge_mlp(ga, gb, row3, starts, ends, e, w1e, w2, b2, g, bt,
              blk_off, n_blk):
    full = lambda shape: pl.BlockSpec(shape, lambda i: (0, 0))
    return pl.pallas_call(
        _edge_body,
        grid=(n_blk,),
        in_specs=[
            pl.BlockSpec((BEDGE, OUT), lambda i: (i, 0)),
            pl.BlockSpec((BEDGE, OUT), lambda i: (i, 0)),
            pl.BlockSpec((1, 1, BEDGE), lambda i: (i + blk_off, 0, 0)),
            pl.BlockSpec((G, 1), lambda i: (0, 0)),
            pl.BlockSpec((G, 1), lambda i: (0, 0)),
            pl.BlockSpec((BEDGE, D_EDGE), lambda i: (i + blk_off, 0)),
            full((D_EDGE, OUT)),
            full((OUT, OUT)),
            full((1, OUT)),
            full((1, OUT)),
            full((1, OUT)),
        ],
        out_specs=[
            pl.BlockSpec((BEDGE, OUT), lambda i: (i, 0)),
            pl.BlockSpec((G, OUT), lambda i: (0, 0)),
        ],
        out_shape=[
            jax.ShapeDtypeStruct((n_blk * BEDGE, OUT), jnp.float32),
            jax.ShapeDtypeStruct((G, OUT), jnp.float32),
        ],
        scratch_shapes=[pltpu.VMEM((G, OUT), jnp.float32)],
        compiler_params=pltpu.CompilerParams(
            dimension_semantics=("arbitrary",)),
    )(ga, gb, row3, starts, ends, e, w1e, w2, b2, g, bt)


def _node_body(x_ref, recv0_ref, recv1_ref, ea_ref, oh_ref, u_ref,
               nw1x_ref, nw1r_ref, nw1u_ref, nb1_ref, nw2_ref, nb2_ref,
               ng_ref, nbt_ref,
               gw1u_ref, gw1n_ref, gw1e_ref, gb1_ref, gw2_ref, gb2_ref,
               gg_ref, gbt_ref,
               xn_ref, un_ref, acc_ref):
    i = pl.program_id(0)

    @pl.when(i == 0)
    def _():
        acc_ref[...] = jnp.zeros_like(acc_ref)

    oh = oh_ref[...]
    un_tab = jnp.dot(u_ref[...], nw1u_ref[...],
                     preferred_element_type=jnp.float32)
    h = jnp.maximum(
        jnp.dot(x_ref[...], nw1x_ref[...], preferred_element_type=jnp.float32)
        + jnp.dot(recv0_ref[...] + recv1_ref[...], nw1r_ref[...],
                  preferred_element_type=jnp.float32)
        + jnp.dot(oh, un_tab, preferred_element_type=jnp.float32)
        + nb1_ref[...], 0.0)
    h2 = jnp.maximum(
        jnp.dot(h, nw2_ref[...], preferred_element_type=jnp.float32)
        + nb2_ref[...], 0.0)
    xn = _ln(h2, ng_ref[...], nbt_ref[...])
    xn_ref[...] = xn

    contract0 = (((0,), (0,)), ((), ()))
    acc_ref[...] += lax.dot_general(
        oh, xn, contract0, preferred_element_type=jnp.float32)

    @pl.when(i == pl.num_programs(0) - 1)
    def _():
        na = acc_ref[...]
        ea = ea_ref[...]
        hg = jnp.maximum(
            jnp.dot(u_ref[...], gw1u_ref[...],
                    preferred_element_type=jnp.float32)
            + jnp.dot(na, gw1n_ref[...], preferred_element_type=jnp.float32)
            + jnp.dot(ea, gw1e_ref[...], preferred_element_type=jnp.float32)
            + gb1_ref[...], 0.0)
        hg2 = jnp.maximum(
            jnp.dot(hg, gw2_ref[...], preferred_element_type=jnp.float32)
            + gb2_ref[...], 0.0)
        un_ref[...] = _ln(hg2, gg_ref[...], gbt_ref[...])


def _node_global(x, recv0, recv1, ea, onehot, u,
                 nw1x, nw1r, nw1u, nb1, nw2, nb2, ng, nbt,
                 gw1u, gw1n, gw1e, gb1, gw2, gb2, gg, gbt):
    grid = N // BN
    full = lambda shape: pl.BlockSpec(shape, lambda i: (0, 0))
    blk = lambda w: pl.BlockSpec((BN, w), lambda i: (i, 0))
    return pl.pallas_call(
        _node_body,
        grid=(grid,),
        in_specs=[
            blk(D_NODE), blk(OUT), blk(OUT),
            full((G, OUT)),
            blk(G),
            full((G, D_NODE)),
            full((D_NODE, OUT)), full((OUT, OUT)), full((D_NODE, OUT)),
            full((1, OUT)), full((OUT, OUT)), full((1, OUT)),
            full((1, OUT)), full((1, OUT)),
            full((D_NODE, OUT)), full((OUT, OUT)), full((OUT, OUT)),
            full((1, OUT)), full((OUT, OUT)), full((1, OUT)),
            full((1, OUT)), full((1, OUT)),
        ],
        out_specs=[
            pl.BlockSpec((BN, OUT), lambda i: (i, 0)),
            pl.BlockSpec((G, OUT), lambda i: (0, 0)),
        ],
        out_shape=[
            jax.ShapeDtypeStruct((N, OUT), jnp.float32),
            jax.ShapeDtypeStruct((G, OUT), jnp.float32),
        ],
        scratch_shapes=[pltpu.VMEM((G, OUT), jnp.float32)],
        compiler_params=pltpu.CompilerParams(
            dimension_semantics=("arbitrary",)),
    )(x, recv0, recv1, ea, onehot, u,
      nw1x, nw1r, nw1u, nb1, nw2, nb2, ng, nbt,
      gw1u, gw1n, gw1e, gb1, gw2, gb2, gg, gbt)


_SC_KB = 80
_SC_SUB = 8
_SC_CHUNK = _SC_KB * _SC_SUB

_NPAD = 10240
_HN = _NPAD // 2
_TRASH = _HN
_ACC = _HN + 8


def _sc_scatter2(e3_parts, ei4, zrows):
    info = plsc.get_sparse_core_info()
    ns = info.num_subcores
    part_rows = [p.shape[0] // _SC_SUB for p in e3_parts]
    stripe = _HN // ns
    mesh = plsc.VectorSubcoreMesh(core_axis_name="c", subcore_axis_name="s")

    @functools.partial(
        pl.kernel, mesh=mesh,
        out_type=jax.ShapeDtypeStruct((_NPAD, OUT), jnp.float32),
        scratch_types=[
            pltpu.VMEM((_SC_SUB, _SC_KB, OUT), jnp.float32),
            pltpu.VMEM((_SC_SUB, _SC_KB), jnp.int32),
            pltpu.VMEM_SHARED((_ACC, OUT), jnp.float32),
            pltpu.SemaphoreType.DMA,
        ],
    )
    def k(*refs):
        e_hbms = refs[:len(e3_parts)]
        ei_hbm, z_hbm, outr_hbm, data_v, idx_v, acc_sh, sem = \
            refs[len(e3_parts):]
        c = lax.axis_index("c")
        s = lax.axis_index("s")
        base = c * _HN

        pltpu.sync_copy(z_hbm.at[pl.ds(0, stripe)],
                        acc_sh.at[pl.ds(s * stripe, stripe)])
        plsc.subcore_barrier()

        def do_row(e_hbm, local, glob):
            cp1 = pltpu.make_async_copy(
                e_hbm.at[pl.ds(local * _SC_SUB, _SC_SUB)], data_v, sem)
            cp1.start()
            cp2 = pltpu.make_async_copy(ei_hbm.at[1, glob], idx_v, sem)
            cp2.start()
            cp1.wait()
            cp2.wait()
            for kk in range(_SC_SUB):
                for jj in range(_SC_KB // 16):
                    v = idx_v[kk, pl.ds(jj * 16, 16)] - base
                    oob = (v < 0) | (v >= _HN)
                    idx_v[kk, pl.ds(jj * 16, 16)] = jnp.where(oob, _TRASH, v)
            for kk in range(_SC_SUB):
                pltpu.sync_copy(data_v.at[kk], acc_sh.at[idx_v.at[kk]],
                                add=True)

        row_lo = 0
        for e_hbm, rows_p in zip(e_hbms, part_rows):
            n_rounds = rows_p // ns
            n_left = rows_p - n_rounds * ns

            def body(j, _, e_hbm=e_hbm, lo=row_lo):
                local = j * ns + s
                do_row(e_hbm, local, lo + local)
                return 0

            lax.fori_loop(0, n_rounds, body, 0)

            @pl.when(s < n_left)
            def _(e_hbm=e_hbm, lo=row_lo, n_rounds=n_rounds):
                local = n_rounds * ns + s
                do_row(e_hbm, local, lo + local)

            row_lo += rows_p

        plsc.subcore_barrier()
        pltpu.sync_copy(acc_sh.at[pl.ds(s * stripe, stripe)],
                        outr_hbm.at[pl.ds(base + s * stripe, stripe)])

    return k(*e3_parts, ei4, zrows)


def _sc_gather2(tab2, gi4, row_lo, row_n):
    info = plsc.get_sparse_core_info()
    ns = info.num_subcores
    n_rounds = row_n // ns
    n_left = row_n - n_rounds * ns
    mesh = plsc.VectorSubcoreMesh(core_axis_name="c", subcore_axis_name="s")

    @functools.partial(
        pl.kernel, mesh=mesh,
        out_type=jax.ShapeDtypeStruct((2, row_n * _SC_SUB, _SC_KB, OUT),
                                      jnp.float32),
        scratch_types=[
            pltpu.VMEM((_SC_SUB, _SC_KB, OUT), jnp.float32),
            pltpu.VMEM((_SC_SUB, _SC_KB), jnp.int32),
            pltpu.SemaphoreType.DMA,
        ],
    )
    def k(tab_hbm, gi_hbm, out_hbm, data_v, idx_v, sem):
        c = lax.axis_index("c")
        s = lax.axis_index("s")

        def do_row(local):
            pltpu.sync_copy(gi_hbm.at[c, row_lo + local], idx_v)
            for kk in range(_SC_SUB):
                pltpu.make_async_copy(
                    tab_hbm.at[idx_v.at[kk]], data_v.at[kk], sem).start()
            for kk in range(_SC_SUB):
                pltpu.make_async_copy(
                    tab_hbm.at[idx_v.at[kk]], data_v.at[kk], sem).wait()
            pltpu.sync_copy(
                data_v, out_hbm.at[c, pl.ds(local * _SC_SUB, _SC_SUB)])

        def body(j, _):
            do_row(j * ns + s)
            return 0

        lax.fori_loop(0, n_rounds, body, 0)

        @pl.when(s < n_left)
        def _():
            do_row(n_rounds * ns + s)

    return k(tab2, gi4)


def kernel(x, e, u, edge_index, batch,
           eW1, eb1, eW2, eb2, eg, ebt,
           nW1, nb1, nW2, nb2, ng, nbt,
           gW1, gb1, gW2, gb2, gg, gbt):
    row = edge_index[0]
    col = edge_index[1]
    onehot = (batch[:, None] == jnp.arange(G, dtype=jnp.int32)[None, :]
              ).astype(jnp.float32)

    r1 = lambda v: v.reshape(1, -1)

    w1e = eW1[:D_EDGE]
    w1r = eW1[D_EDGE:D_EDGE + D_NODE]
    w1s = eW1[D_EDGE + D_NODE:D_EDGE + 2 * D_NODE]
    w1g = eW1[D_EDGE + 2 * D_NODE:]

    ab_tab = _prep_tables(x, onehot, u, w1r, w1s, w1g, r1(eb1))
    tab2 = ab_tab.reshape(2 * N, OUT)
    gi4 = jnp.stack([col, row + N]).reshape(2, E // _SC_CHUNK,
                                            _SC_SUB, _SC_KB)
    cnt = jnp.sum(onehot, axis=0).astype(jnp.int32)
    ends_f = jnp.cumsum(cnt)
    starts = (ends_f - cnt).reshape(G, 1)
    ends = ends_f.reshape(G, 1)
    row3 = row.reshape(E // BEDGE, 1, BEDGE)

    n_rows_total = E // _SC_CHUNK
    half_rows = n_rows_total // 2
    half_blk = (half_rows * _SC_CHUNK) // BEDGE

    e_parts, ea_parts = [], []
    for p in range(2):
        g2 = _sc_gather2(tab2, gi4, p * half_rows, half_rows)
        ga = g2[0].reshape(half_rows * _SC_CHUNK, OUT)
        gb = g2[1].reshape(half_rows * _SC_CHUNK, OUT)
        en_p, ea_p = _edge_mlp(ga, gb, row3, starts, ends, e, w1e, eW2,
                               r1(eb2), r1(eg), r1(ebt),
                               p * half_blk, half_blk)
        e_parts.append(en_p)
        ea_parts.append(ea_p)

    ea = ea_parts[0] + ea_parts[1]
    ei4 = edge_index.reshape(2, E // _SC_CHUNK, _SC_SUB, _SC_KB)
    zrows = jnp.zeros((320, OUT), jnp.float32)
    recvs = []
    for p in range(2):
        e3_p = e_parts[p].reshape(half_rows * _SC_SUB, _SC_KB, OUT)
        recvs.append(_sc_scatter2([e3_p], ei4[:, p * half_rows:
                                               (p + 1) * half_rows],
                                  zrows))
    e_new = jnp.concatenate(e_parts, axis=0)

    x_new, u_new = _node_global(
        x, recvs[0], recvs[1], ea, onehot, u,
        nW1[:D_NODE], nW1[D_NODE:D_NODE + OUT], nW1[D_NODE + OUT:],
        r1(nb1), nW2, r1(nb2), r1(ng), r1(nbt),
        gW1[:D_NODE], gW1[D_NODE:D_NODE + OUT], gW1[D_NODE + OUT:],
        r1(gb1), gW2, r1(gb2), r1(gg), r1(gbt))

    return (x_new, e_new, u_new)


# speedup vs baseline: 1.1254x; 1.0328x over previous
import functools

import jax
import jax.numpy as jnp
from jax import lax
from jax.experimental import pallas as pl
from jax.experimental.pallas import tpu as pltpu
from jax.experimental.pallas import tpu_sc as plsc

N = 10000
E = 320000
D_NODE = 128
D_EDGE = 16
G = 16
OUT = 128

BN = 2000
BEDGE = 2000


def _ln(h, g, b):
    mu = jnp.mean(h, axis=-1, keepdims=True)
    var = jnp.mean((h - mu) * (h - mu), axis=-1, keepdims=True)
    return (h - mu) * lax.rsqrt(var + 1e-5) * g + b


def _prep_body(x_ref, oh_ref, u_ref, wr_ref, ws_ref, wg_ref, b1_ref,
               ab_ref):
    xb = x_ref[...]
    ab_ref[0] = jnp.dot(xb, wr_ref[...], preferred_element_type=jnp.float32)
    ug = jnp.dot(u_ref[...], wg_ref[...], preferred_element_type=jnp.float32)
    ab_ref[1] = (jnp.dot(xb, ws_ref[...], preferred_element_type=jnp.float32)
                 + jnp.dot(oh_ref[...], ug, preferred_element_type=jnp.float32)
                 + b1_ref[...])


def _prep_tables(x, onehot, u, wr, ws, wg, b1):
    grid = N // BN
    full = lambda shape: pl.BlockSpec(shape, lambda i: (0, 0))
    return pl.pallas_call(
        _prep_body,
        grid=(grid,),
        in_specs=[
            pl.BlockSpec((BN, D_NODE), lambda i: (i, 0)),
            pl.BlockSpec((BN, G), lambda i: (i, 0)),
            full((G, D_NODE)),
            full((D_NODE, OUT)),
            full((D_NODE, OUT)),
            full((D_NODE, OUT)),
            full((1, OUT)),
        ],
        out_specs=pl.BlockSpec((2, BN, OUT), lambda i: (0, i, 0)),
        out_shape=jax.ShapeDtypeStruct((2, N, OUT), jnp.float32),
        compiler_params=pltpu.CompilerParams(
            dimension_semantics=("arbitrary",)),
    )(x, onehot, u, wr, ws, wg, b1)


def _edge_body(ga_ref, gb_ref, row_ref, st_ref, en_ref, e_ref, w1e_ref,
               w2_ref, b2_ref, g_ref, bt_ref, out_ref, ea_ref, acc_ref):
    i = pl.program_id(0)

    @pl.when(i == 0)
    def _():
        acc_ref[...] = jnp.zeros_like(acc_ref)

    h1 = jnp.maximum(
        ga_ref[...] + gb_ref[...]
        + jnp.dot(e_ref[...], w1e_ref[...],
                  preferred_element_type=jnp.float32), 0.0)
    h2 = jnp.maximum(
        jnp.dot(h1, w2_ref[...], preferred_element_type=jnp.float32)
        + b2_ref[...], 0.0)
    en = _ln(h2, g_ref[...], bt_ref[...])
    out_ref[...] = en
    rr = row_ref[0]
    ohgT = jnp.logical_and(rr >= st_ref[...], rr < en_ref[...]
                           ).astype(jnp.float32)
    acc_ref[...] += lax.dot_general(
        ohgT, en, (((1,), (0,)), ((), ())),
        preferred_element_type=jnp.float32)

    @pl.when(i == pl.num_programs(0) - 1)
    def _():
        ea_ref[...] = acc_ref[...]


def _edge_mlp(ga, gb, row3, starts, ends, e, w1e, w2, b2, g, bt,
              blk_off, n_blk):
    full = lambda shape: pl.BlockSpec(shape, lambda i: (0, 0))
    return pl.pallas_call(
        _edge_body,
        grid=(n_blk,),
        in_specs=[
            pl.BlockSpec((BEDGE, OUT), lambda i: (i, 0)),
            pl.BlockSpec((BEDGE, OUT), lambda i: (i, 0)),
            pl.BlockSpec((1, 1, BEDGE), lambda i: (i + blk_off, 0, 0)),
            pl.BlockSpec((G, 1), lambda i: (0, 0)),
            pl.BlockSpec((G, 1), lambda i: (0, 0)),
            pl.BlockSpec((BEDGE, D_EDGE), lambda i: (i + blk_off, 0)),
            full((D_EDGE, OUT)),
            full((OUT, OUT)),
            full((1, OUT)),
            full((1, OUT)),
            full((1, OUT)),
        ],
        out_specs=[
            pl.BlockSpec((BEDGE, OUT), lambda i: (i, 0)),
            pl.BlockSpec((G, OUT), lambda i: (0, 0)),
        ],
        out_shape=[
            jax.ShapeDtypeStruct((n_blk * BEDGE, OUT), jnp.float32),
            jax.ShapeDtypeStruct((G, OUT), jnp.float32),
        ],
        scratch_shapes=[pltpu.VMEM((G, OUT), jnp.float32)],
        compiler_params=pltpu.CompilerParams(
            dimension_semantics=("arbitrary",)),
    )(ga, gb, row3, starts, ends, e, w1e, w2, b2, g, bt)


def _node_body(x_ref, recv0_ref, recv1_ref, ea_ref, oh_ref, u_ref,
               nw1x_ref, nw1r_ref, nw1u_ref, nb1_ref, nw2_ref, nb2_ref,
               ng_ref, nbt_ref,
               gw1u_ref, gw1n_ref, gw1e_ref, gb1_ref, gw2_ref, gb2_ref,
               gg_ref, gbt_ref,
               xn_ref, un_ref, acc_ref):
    i = pl.program_id(0)

    @pl.when(i == 0)
    def _():
        acc_ref[...] = jnp.zeros_like(acc_ref)

    oh = oh_ref[...]
    un_tab = jnp.dot(u_ref[...], nw1u_ref[...],
                     preferred_element_type=jnp.float32)
    h = jnp.maximum(
        jnp.dot(x_ref[...], nw1x_ref[...], preferred_element_type=jnp.float32)
        + jnp.dot(recv0_ref[...] + recv1_ref[...], nw1r_ref[...],
                  preferred_element_type=jnp.float32)
        + jnp.dot(oh, un_tab, preferred_element_type=jnp.float32)
        + nb1_ref[...], 0.0)
    h2 = jnp.maximum(
        jnp.dot(h, nw2_ref[...], preferred_element_type=jnp.float32)
        + nb2_ref[...], 0.0)
    xn = _ln(h2, ng_ref[...], nbt_ref[...])
    xn_ref[...] = xn

    contract0 = (((0,), (0,)), ((), ()))
    acc_ref[...] += lax.dot_general(
        oh, xn, contract0, preferred_element_type=jnp.float32)

    @pl.when(i == pl.num_programs(0) - 1)
    def _():
        na = acc_ref[...]
        ea = ea_ref[...]
        hg = jnp.maximum(
            jnp.dot(u_ref[...], gw1u_ref[...],
                    preferred_element_type=jnp.float32)
            + jnp.dot(na, gw1n_ref[...], preferred_element_type=jnp.float32)
            + jnp.dot(ea, gw1e_ref[...], preferred_element_type=jnp.float32)
            + gb1_ref[...], 0.0)
        hg2 = jnp.maximum(
            jnp.dot(hg, gw2_ref[...], preferred_element_type=jnp.float32)
            + gb2_ref[...], 0.0)
        un_ref[...] = _ln(hg2, gg_ref[...], gbt_ref[...])


def _node_global(x, recv0, recv1, ea, onehot, u,
                 nw1x, nw1r, nw1u, nb1, nw2, nb2, ng, nbt,
                 gw1u, gw1n, gw1e, gb1, gw2, gb2, gg, gbt):
    grid = N // BN
    full = lambda shape: pl.BlockSpec(shape, lambda i: (0, 0))
    blk = lambda w: pl.BlockSpec((BN, w), lambda i: (i, 0))
    return pl.pallas_call(
        _node_body,
        grid=(grid,),
        in_specs=[
            blk(D_NODE), blk(OUT), blk(OUT),
            full((G, OUT)),
            blk(G),
            full((G, D_NODE)),
            full((D_NODE, OUT)), full((OUT, OUT)), full((D_NODE, OUT)),
            full((1, OUT)), full((OUT, OUT)), full((1, OUT)),
            full((1, OUT)), full((1, OUT)),
            full((D_NODE, OUT)), full((OUT, OUT)), full((OUT, OUT)),
            full((1, OUT)), full((OUT, OUT)), full((1, OUT)),
            full((1, OUT)), full((1, OUT)),
        ],
        out_specs=[
            pl.BlockSpec((BN, OUT), lambda i: (i, 0)),
            pl.BlockSpec((G, OUT), lambda i: (0, 0)),
        ],
        out_shape=[
            jax.ShapeDtypeStruct((N, OUT), jnp.float32),
            jax.ShapeDtypeStruct((G, OUT), jnp.float32),
        ],
        scratch_shapes=[pltpu.VMEM((G, OUT), jnp.float32)],
        compiler_params=pltpu.CompilerParams(
            dimension_semantics=("arbitrary",)),
    )(x, recv0, recv1, ea, onehot, u,
      nw1x, nw1r, nw1u, nb1, nw2, nb2, ng, nbt,
      gw1u, gw1n, gw1e, gb1, gw2, gb2, gg, gbt)


_SC_KB = 80
_SC_SUB = 8
_SC_CHUNK = _SC_KB * _SC_SUB

_NPAD = 10240
_HN = _NPAD // 2
_TRASH = _HN
_ACC = _HN + 8


def _sc_scatter2(e3_parts, ei4, zrows):
    info = plsc.get_sparse_core_info()
    ns = info.num_subcores
    part_rows = [p.shape[0] // _SC_SUB for p in e3_parts]
    stripe = _HN // ns
    mesh = plsc.VectorSubcoreMesh(core_axis_name="c", subcore_axis_name="s")

    @functools.partial(
        pl.kernel, mesh=mesh,
        out_type=jax.ShapeDtypeStruct((_NPAD, OUT), jnp.float32),
        scratch_types=[
            pltpu.VMEM((_SC_SUB, _SC_KB, OUT), jnp.float32),
            pltpu.VMEM((_SC_SUB, _SC_KB), jnp.int32),
            pltpu.VMEM_SHARED((_ACC, OUT), jnp.float32),
            pltpu.SemaphoreType.DMA,
        ],
    )
    def k(*refs):
        e_hbms = refs[:len(e3_parts)]
        ei_hbm, z_hbm, outr_hbm, data_v, idx_v, acc_sh, sem = \
            refs[len(e3_parts):]
        c = lax.axis_index("c")
        s = lax.axis_index("s")
        base = c * _HN

        pltpu.sync_copy(z_hbm.at[pl.ds(0, stripe)],
                        acc_sh.at[pl.ds(s * stripe, stripe)])
        plsc.subcore_barrier()

        def do_row(e_hbm, local, glob):
            cp1 = pltpu.make_async_copy(
                e_hbm.at[pl.ds(local * _SC_SUB, _SC_SUB)], data_v, sem)
            cp1.start()
            cp2 = pltpu.make_async_copy(ei_hbm.at[1, glob], idx_v, sem)
            cp2.start()
            cp1.wait()
            cp2.wait()
            for kk in range(_SC_SUB):
                for jj in range(_SC_KB // 16):
                    v = idx_v[kk, pl.ds(jj * 16, 16)] - base
                    oob = (v < 0) | (v >= _HN)
                    idx_v[kk, pl.ds(jj * 16, 16)] = jnp.where(oob, _TRASH, v)
            for kk in range(_SC_SUB):
                pltpu.sync_copy(data_v.at[kk], acc_sh.at[idx_v.at[kk]],
                                add=True)

        row_lo = 0
        for e_hbm, rows_p in zip(e_hbms, part_rows):
            n_rounds = rows_p // ns
            n_left = rows_p - n_rounds * ns

            def body(j, _, e_hbm=e_hbm, lo=row_lo):
                local = j * ns + s
                do_row(e_hbm, local, lo + local)
                return 0

            lax.fori_loop(0, n_rounds, body, 0)

            @pl.when(s < n_left)
            def _(e_hbm=e_hbm, lo=row_lo, n_rounds=n_rounds):
                local = n_rounds * ns + s
                do_row(e_hbm, local, lo + local)

            row_lo += rows_p

        plsc.subcore_barrier()
        pltpu.sync_copy(acc_sh.at[pl.ds(s * stripe, stripe)],
                        outr_hbm.at[pl.ds(base + s * stripe, stripe)])

    return k(*e3_parts, ei4, zrows)


def _sc_gather2(tab2, gi4, row_lo, row_n):
    info = plsc.get_sparse_core_info()
    ns = info.num_subcores
    n_rounds = row_n // ns
    n_left = row_n - n_rounds * ns
    mesh = plsc.VectorSubcoreMesh(core_axis_name="c", subcore_axis_name="s")

    @functools.partial(
        pl.kernel, mesh=mesh,
        out_type=jax.ShapeDtypeStruct((2, row_n * _SC_SUB, _SC_KB, OUT),
                                      jnp.float32),
        scratch_types=[
            pltpu.VMEM((_SC_SUB, _SC_KB, OUT), jnp.float32),
            pltpu.VMEM((_SC_SUB, _SC_KB), jnp.int32),
            pltpu.SemaphoreType.DMA,
        ],
    )
    def k(tab_hbm, gi_hbm, out_hbm, data_v, idx_v, sem):
        c = lax.axis_index("c")
        s = lax.axis_index("s")

        def do_row(local):
            pltpu.sync_copy(gi_hbm.at[c, row_lo + local], idx_v)
            for kk in range(_SC_SUB):
                pltpu.make_async_copy(
                    tab_hbm.at[idx_v.at[kk]], data_v.at[kk], sem).start()
            for kk in range(_SC_SUB):
                pltpu.make_async_copy(
                    tab_hbm.at[idx_v.at[kk]], data_v.at[kk], sem).wait()
            pltpu.sync_copy(
                data_v, out_hbm.at[c, pl.ds(local * _SC_SUB, _SC_SUB)])

        def body(j, _):
            do_row(j * ns + s)
            return 0

        lax.fori_loop(0, n_rounds, body, 0)

        @pl.when(s < n_left)
        def _():
            do_row(n_rounds * ns + s)

    return k(tab2, gi4)


def kernel(x, e, u, edge_index, batch,
           eW1, eb1, eW2, eb2, eg, ebt,
           nW1, nb1, nW2, nb2, ng, nbt,
           gW1, gb1, gW2, gb2, gg, gbt):
    row = edge_index[0]
    col = edge_index[1]
    onehot = (batch[:, None] == jnp.arange(G, dtype=jnp.int32)[None, :]
              ).astype(jnp.float32)

    r1 = lambda v: v.reshape(1, -1)

    w1e = eW1[:D_EDGE]
    w1r = eW1[D_EDGE:D_EDGE + D_NODE]
    w1s = eW1[D_EDGE + D_NODE:D_EDGE + 2 * D_NODE]
    w1g = eW1[D_EDGE + 2 * D_NODE:]

    ab_tab = _prep_tables(x, onehot, u, w1r, w1s, w1g, r1(eb1))
    tab2 = ab_tab.reshape(2 * N, OUT)
    gi4 = jnp.stack([col, row + N]).reshape(2, E // _SC_CHUNK,
                                            _SC_SUB, _SC_KB)
    cnt = jnp.sum(onehot, axis=0).astype(jnp.int32)
    ends_f = jnp.cumsum(cnt)
    starts = (ends_f - cnt).reshape(G, 1)
    ends = ends_f.reshape(G, 1)
    row3 = row.reshape(E // BEDGE, 1, BEDGE)

    K = 4
    n_rows_total = E // _SC_CHUNK
    part_rows = n_rows_total // K
    part_blk = (part_rows * _SC_CHUNK) // BEDGE
    ei4 = edge_index.reshape(2, E // _SC_CHUNK, _SC_SUB, _SC_KB)
    zrows = jnp.zeros((320, OUT), jnp.float32)

    e_parts, ea_parts, recvs = [], [], []
    for p in range(K):
        g2 = _sc_gather2(tab2, gi4, p * part_rows, part_rows)
        ga = g2[0].reshape(part_rows * _SC_CHUNK, OUT)
        gb = g2[1].reshape(part_rows * _SC_CHUNK, OUT)
        en_p, ea_p = _edge_mlp(ga, gb, row3, starts, ends, e, w1e, eW2,
                               r1(eb2), r1(eg), r1(ebt),
                               p * part_blk, part_blk)
        e_parts.append(en_p)
        ea_parts.append(ea_p)
        e3_p = en_p.reshape(part_rows * _SC_SUB, _SC_KB, OUT)
        recvs.append(_sc_scatter2([e3_p], ei4[:, p * part_rows:
                                               (p + 1) * part_rows],
                                  zrows))

    ea = sum(ea_parts[1:], ea_parts[0])
    recv_a = recvs[0] + recvs[1]
    recv_b = recvs[2] + recvs[3]
    e_new = jnp.concatenate(e_parts, axis=0)

    x_new, u_new = _node_global(
        x, recv_a, recv_b, ea, onehot, u,
        nW1[:D_NODE], nW1[D_NODE:D_NODE + OUT], nW1[D_NODE + OUT:],
        r1(nb1), nW2, r1(nb2), r1(ng), r1(nbt),
        gW1[:D_NODE], gW1[D_NODE:D_NODE + OUT], gW1[D_NODE + OUT:],
        r1(gb1), gW2, r1(gb2), r1(gg), r1(gbt))

    return (x_new, e_new, u_new)


# speedup vs baseline: 1.1532x; 1.0247x over previous
import functools

import jax
import jax.numpy as jnp
from jax import lax
from jax.experimental import pallas as pl
from jax.experimental.pallas import tpu as pltpu
from jax.experimental.pallas import tpu_sc as plsc

N = 10000
E = 320000
D_NODE = 128
D_EDGE = 16
G = 16
OUT = 128

BN = 2000
BEDGE = 4000


def _ln(h, g, b):
    mu = jnp.mean(h, axis=-1, keepdims=True)
    var = jnp.mean((h - mu) * (h - mu), axis=-1, keepdims=True)
    return (h - mu) * lax.rsqrt(var + 1e-5) * g + b


def _prep_body(x_ref, oh_ref, u_ref, wr_ref, ws_ref, wg_ref, b1_ref,
               ab_ref):
    xb = x_ref[...]
    ab_ref[0] = jnp.dot(xb, wr_ref[...], preferred_element_type=jnp.float32)
    ug = jnp.dot(u_ref[...], wg_ref[...], preferred_element_type=jnp.float32)
    ab_ref[1] = (jnp.dot(xb, ws_ref[...], preferred_element_type=jnp.float32)
                 + jnp.dot(oh_ref[...], ug, preferred_element_type=jnp.float32)
                 + b1_ref[...])


def _prep_tables(x, onehot, u, wr, ws, wg, b1):
    grid = N // BN
    full = lambda shape: pl.BlockSpec(shape, lambda i: (0, 0))
    return pl.pallas_call(
        _prep_body,
        grid=(grid,),
        in_specs=[
            pl.BlockSpec((BN, D_NODE), lambda i: (i, 0)),
            pl.BlockSpec((BN, G), lambda i: (i, 0)),
            full((G, D_NODE)),
            full((D_NODE, OUT)),
            full((D_NODE, OUT)),
            full((D_NODE, OUT)),
            full((1, OUT)),
        ],
        out_specs=pl.BlockSpec((2, BN, OUT), lambda i: (0, i, 0)),
        out_shape=jax.ShapeDtypeStruct((2, N, OUT), jnp.float32),
        compiler_params=pltpu.CompilerParams(
            dimension_semantics=("arbitrary",)),
    )(x, onehot, u, wr, ws, wg, b1)


def _edge_body(ga_ref, gb_ref, row_ref, st_ref, en_ref, e_ref, w1e_ref,
               w2_ref, b2_ref, g_ref, bt_ref, out_ref, ea_ref, acc_ref):
    i = pl.program_id(0)

    @pl.when(i == 0)
    def _():
        acc_ref[...] = jnp.zeros_like(acc_ref)

    h1 = jnp.maximum(
        ga_ref[...] + gb_ref[...]
        + jnp.dot(e_ref[...], w1e_ref[...],
                  preferred_element_type=jnp.float32), 0.0)
    h2 = jnp.maximum(
        jnp.dot(h1, w2_ref[...], preferred_element_type=jnp.float32)
        + b2_ref[...], 0.0)
    en = _ln(h2, g_ref[...], bt_ref[...])
    out_ref[...] = en
    rr = row_ref[0]
    ohgT = jnp.logical_and(rr >= st_ref[...], rr < en_ref[...]
                           ).astype(jnp.float32)
    acc_ref[...] += lax.dot_general(
        ohgT, en, (((1,), (0,)), ((), ())),
        preferred_element_type=jnp.float32)

    @pl.when(i == pl.num_programs(0) - 1)
    def _():
        ea_ref[...] = acc_ref[...]


def _edge_mlp(ga, gb, row3, starts, ends, e, w1e, w2, b2, g, bt,
              blk_off, n_blk):
    full = lambda shape: pl.BlockSpec(shape, lambda i: (0, 0))
    return pl.pallas_call(
        _edge_body,
        grid=(n_blk,),
        in_specs=[
            pl.BlockSpec((BEDGE, OUT), lambda i: (i, 0)),
            pl.BlockSpec((BEDGE, OUT), lambda i: (i, 0)),
            pl.BlockSpec((1, 1, BEDGE), lambda i: (i + blk_off, 0, 0)),
            pl.BlockSpec((G, 1), lambda i: (0, 0)),
            pl.BlockSpec((G, 1), lambda i: (0, 0)),
            pl.BlockSpec((BEDGE, D_EDGE), lambda i: (i + blk_off, 0)),
            full((D_EDGE, OUT)),
            full((OUT, OUT)),
            full((1, OUT)),
            full((1, OUT)),
            full((1, OUT)),
        ],
        out_specs=[
            pl.BlockSpec((BEDGE, OUT), lambda i: (i, 0)),
            pl.BlockSpec((G, OUT), lambda i: (0, 0)),
        ],
        out_shape=[
            jax.ShapeDtypeStruct((n_blk * BEDGE, OUT), jnp.float32),
            jax.ShapeDtypeStruct((G, OUT), jnp.float32),
        ],
        scratch_shapes=[pltpu.VMEM((G, OUT), jnp.float32)],
        compiler_params=pltpu.CompilerParams(
            dimension_semantics=("arbitrary",)),
    )(ga, gb, row3, starts, ends, e, w1e, w2, b2, g, bt)


def _node_body(x_ref, recv0_ref, recv1_ref, ea_ref, oh_ref, u_ref,
               nw1x_ref, nw1r_ref, nw1u_ref, nb1_ref, nw2_ref, nb2_ref,
               ng_ref, nbt_ref,
               gw1u_ref, gw1n_ref, gw1e_ref, gb1_ref, gw2_ref, gb2_ref,
               gg_ref, gbt_ref,
               xn_ref, un_ref, acc_ref):
    i = pl.program_id(0)

    @pl.when(i == 0)
    def _():
        acc_ref[...] = jnp.zeros_like(acc_ref)

    oh = oh_ref[...]
    un_tab = jnp.dot(u_ref[...], nw1u_ref[...],
                     preferred_element_type=jnp.float32)
    h = jnp.maximum(
        jnp.dot(x_ref[...], nw1x_ref[...], preferred_element_type=jnp.float32)
        + jnp.dot(recv0_ref[...] + recv1_ref[...], nw1r_ref[...],
                  preferred_element_type=jnp.float32)
        + jnp.dot(oh, un_tab, preferred_element_type=jnp.float32)
        + nb1_ref[...], 0.0)
    h2 = jnp.maximum(
        jnp.dot(h, nw2_ref[...], preferred_element_type=jnp.float32)
        + nb2_ref[...], 0.0)
    xn = _ln(h2, ng_ref[...], nbt_ref[...])
    xn_ref[...] = xn

    contract0 = (((0,), (0,)), ((), ()))
    acc_ref[...] += lax.dot_general(
        oh, xn, contract0, preferred_element_type=jnp.float32)

    @pl.when(i == pl.num_programs(0) - 1)
    def _():
        na = acc_ref[...]
        ea = ea_ref[...]
        hg = jnp.maximum(
            jnp.dot(u_ref[...], gw1u_ref[...],
                    preferred_element_type=jnp.float32)
            + jnp.dot(na, gw1n_ref[...], preferred_element_type=jnp.float32)
            + jnp.dot(ea, gw1e_ref[...], preferred_element_type=jnp.float32)
            + gb1_ref[...], 0.0)
        hg2 = jnp.maximum(
            jnp.dot(hg, gw2_ref[...], preferred_element_type=jnp.float32)
            + gb2_ref[...], 0.0)
        un_ref[...] = _ln(hg2, gg_ref[...], gbt_ref[...])


def _node_global(x, recv0, recv1, ea, onehot, u,
                 nw1x, nw1r, nw1u, nb1, nw2, nb2, ng, nbt,
                 gw1u, gw1n, gw1e, gb1, gw2, gb2, gg, gbt):
    grid = N // BN
    full = lambda shape: pl.BlockSpec(shape, lambda i: (0, 0))
    blk = lambda w: pl.BlockSpec((BN, w), lambda i: (i, 0))
    return pl.pallas_call(
        _node_body,
        grid=(grid,),
        in_specs=[
            blk(D_NODE), blk(OUT), blk(OUT),
            full((G, OUT)),
            blk(G),
            full((G, D_NODE)),
            full((D_NODE, OUT)), full((OUT, OUT)), full((D_NODE, OUT)),
            full((1, OUT)), full((OUT, OUT)), full((1, OUT)),
            full((1, OUT)), full((1, OUT)),
            full((D_NODE, OUT)), full((OUT, OUT)), full((OUT, OUT)),
            full((1, OUT)), full((OUT, OUT)), full((1, OUT)),
            full((1, OUT)), full((1, OUT)),
        ],
        out_specs=[
            pl.BlockSpec((BN, OUT), lambda i: (i, 0)),
            pl.BlockSpec((G, OUT), lambda i: (0, 0)),
        ],
        out_shape=[
            jax.ShapeDtypeStruct((N, OUT), jnp.float32),
            jax.ShapeDtypeStruct((G, OUT), jnp.float32),
        ],
        scratch_shapes=[pltpu.VMEM((G, OUT), jnp.float32)],
        compiler_params=pltpu.CompilerParams(
            dimension_semantics=("arbitrary",)),
    )(x, recv0, recv1, ea, onehot, u,
      nw1x, nw1r, nw1u, nb1, nw2, nb2, ng, nbt,
      gw1u, gw1n, gw1e, gb1, gw2, gb2, gg, gbt)


_SC_KB = 80
_SC_SUB = 8
_SC_CHUNK = _SC_KB * _SC_SUB

_NPAD = 10240
_HN = _NPAD // 2
_TRASH = _HN
_ACC = _HN + 8


def _sc_scatter2(e3_parts, ei4, zrows):
    info = plsc.get_sparse_core_info()
    ns = info.num_subcores
    part_rows = [p.shape[0] // _SC_SUB for p in e3_parts]
    stripe = _HN // ns
    mesh = plsc.VectorSubcoreMesh(core_axis_name="c", subcore_axis_name="s")

    @functools.partial(
        pl.kernel, mesh=mesh,
        out_type=jax.ShapeDtypeStruct((_NPAD, OUT), jnp.float32),
        scratch_types=[
            pltpu.VMEM((_SC_SUB, _SC_KB, OUT), jnp.float32),
            pltpu.VMEM((_SC_SUB, _SC_KB), jnp.int32),
            pltpu.VMEM_SHARED((_ACC, OUT), jnp.float32),
            pltpu.SemaphoreType.DMA,
        ],
    )
    def k(*refs):
        e_hbms = refs[:len(e3_parts)]
        ei_hbm, z_hbm, outr_hbm, data_v, idx_v, acc_sh, sem = \
            refs[len(e3_parts):]
        c = lax.axis_index("c")
        s = lax.axis_index("s")
        base = c * _HN

        pltpu.sync_copy(z_hbm.at[pl.ds(0, stripe)],
                        acc_sh.at[pl.ds(s * stripe, stripe)])
        plsc.subcore_barrier()

        def do_row(e_hbm, local, glob):
            cp1 = pltpu.make_async_copy(
                e_hbm.at[pl.ds(local * _SC_SUB, _SC_SUB)], data_v, sem)
            cp1.start()
            cp2 = pltpu.make_async_copy(ei_hbm.at[1, glob], idx_v, sem)
            cp2.start()
            cp1.wait()
            cp2.wait()
            for kk in range(_SC_SUB):
                for jj in range(_SC_KB // 16):
                    v = idx_v[kk, pl.ds(jj * 16, 16)] - base
                    oob = (v < 0) | (v >= _HN)
                    idx_v[kk, pl.ds(jj * 16, 16)] = jnp.where(oob, _TRASH, v)
            for kk in range(_SC_SUB):
                pltpu.sync_copy(data_v.at[kk], acc_sh.at[idx_v.at[kk]],
                                add=True)

        row_lo = 0
        for e_hbm, rows_p in zip(e_hbms, part_rows):
            n_rounds = rows_p // ns
            n_left = rows_p - n_rounds * ns

            def body(j, _, e_hbm=e_hbm, lo=row_lo):
                local = j * ns + s
                do_row(e_hbm, local, lo + local)
                return 0

            lax.fori_loop(0, n_rounds, body, 0)

            @pl.when(s < n_left)
            def _(e_hbm=e_hbm, lo=row_lo, n_rounds=n_rounds):
                local = n_rounds * ns + s
                do_row(e_hbm, local, lo + local)

            row_lo += rows_p

        plsc.subcore_barrier()
        pltpu.sync_copy(acc_sh.at[pl.ds(s * stripe, stripe)],
                        outr_hbm.at[pl.ds(base + s * stripe, stripe)])

    return k(*e3_parts, ei4, zrows)


def _sc_gather2(tab2, gi4, row_lo, row_n):
    info = plsc.get_sparse_core_info()
    ns = info.num_subcores
    n_rounds = row_n // ns
    n_left = row_n - n_rounds * ns
    mesh = plsc.VectorSubcoreMesh(core_axis_name="c", subcore_axis_name="s")

    @functools.partial(
        pl.kernel, mesh=mesh,
        out_type=jax.ShapeDtypeStruct((2, row_n * _SC_SUB, _SC_KB, OUT),
                                      jnp.float32),
        scratch_types=[
            pltpu.VMEM((_SC_SUB, _SC_KB, OUT), jnp.float32),
            pltpu.VMEM((_SC_SUB, _SC_KB), jnp.int32),
            pltpu.SemaphoreType.DMA,
        ],
    )
    def k(tab_hbm, gi_hbm, out_hbm, data_v, idx_v, sem):
        c = lax.axis_index("c")
        s = lax.axis_index("s")

        def do_row(local):
            pltpu.sync_copy(gi_hbm.at[c, row_lo + local], idx_v)
            for kk in range(_SC_SUB):
                pltpu.make_async_copy(
                    tab_hbm.at[idx_v.at[kk]], data_v.at[kk], sem).start()
            for kk in range(_SC_SUB):
                pltpu.make_async_copy(
                    tab_hbm.at[idx_v.at[kk]], data_v.at[kk], sem).wait()
            pltpu.sync_copy(
                data_v, out_hbm.at[c, pl.ds(local * _SC_SUB, _SC_SUB)])

        def body(j, _):
            do_row(j * ns + s)
            return 0

        lax.fori_loop(0, n_rounds, body, 0)

        @pl.when(s < n_left)
        def _():
            do_row(n_rounds * ns + s)

    return k(tab2, gi4)


def kernel(x, e, u, edge_index, batch,
           eW1, eb1, eW2, eb2, eg, ebt,
           nW1, nb1, nW2, nb2, ng, nbt,
           gW1, gb1, gW2, gb2, gg, gbt):
    row = edge_index[0]
    col = edge_index[1]
    onehot = (batch[:, None] == jnp.arange(G, dtype=jnp.int32)[None, :]
              ).astype(jnp.float32)

    r1 = lambda v: v.reshape(1, -1)

    w1e = eW1[:D_EDGE]
    w1r = eW1[D_EDGE:D_EDGE + D_NODE]
    w1s = eW1[D_EDGE + D_NODE:D_EDGE + 2 * D_NODE]
    w1g = eW1[D_EDGE + 2 * D_NODE:]

    ab_tab = _prep_tables(x, onehot, u, w1r, w1s, w1g, r1(eb1))
    tab2 = ab_tab.reshape(2 * N, OUT)
    gi4 = jnp.stack([col, row + N]).reshape(2, E // _SC_CHUNK,
                                            _SC_SUB, _SC_KB)
    cnt = jnp.sum(onehot, axis=0).astype(jnp.int32)
    ends_f = jnp.cumsum(cnt)
    starts = (ends_f - cnt).reshape(G, 1)
    ends = ends_f.reshape(G, 1)
    row3 = row.reshape(E // BEDGE, 1, BEDGE)

    K = 4
    n_rows_total = E // _SC_CHUNK
    part_rows = n_rows_total // K
    part_blk = (part_rows * _SC_CHUNK) // BEDGE
    ei4 = edge_index.reshape(2, E // _SC_CHUNK, _SC_SUB, _SC_KB)
    zrows = jnp.zeros((320, OUT), jnp.float32)

    e_parts, ea_parts, recvs = [], [], []
    for p in range(K):
        g2 = _sc_gather2(tab2, gi4, p * part_rows, part_rows)
        ga = g2[0].reshape(part_rows * _SC_CHUNK, OUT)
        gb = g2[1].reshape(part_rows * _SC_CHUNK, OUT)
        en_p, ea_p = _edge_mlp(ga, gb, row3, starts, ends, e, w1e, eW2,
                               r1(eb2), r1(eg), r1(ebt),
                               p * part_blk, part_blk)
        e_parts.append(en_p)
        ea_parts.append(ea_p)
        e3_p = en_p.reshape(part_rows * _SC_SUB, _SC_KB, OUT)
        recvs.append(_sc_scatter2([e3_p], ei4[:, p * part_rows:
                                               (p + 1) * part_rows],
                                  zrows))

    ea = sum(ea_parts[1:], ea_parts[0])
    recv_a = recvs[0] + recvs[1]
    recv_b = recvs[2] + recvs[3]
    e_new = jnp.concatenate(e_parts, axis=0)

    x_new, u_new = _node_global(
        x, recv_a, recv_b, ea, onehot, u,
        nW1[:D_NODE], nW1[D_NODE:D_NODE + OUT], nW1[D_NODE + OUT:],
        r1(nb1), nW2, r1(nb2), r1(ng), r1(nbt),
        gW1[:D_NODE], gW1[D_NODE:D_NODE + OUT], gW1[D_NODE + OUT:],
        r1(gb1), gW2, r1(gb2), r1(gg), r1(gbt))

    return (x_new, e_new, u_new)


# speedup vs baseline: 1.1563x; 1.0027x over previous
import functools

import jax
import jax.numpy as jnp
from jax import lax
from jax.experimental import pallas as pl
from jax.experimental.pallas import tpu as pltpu
from jax.experimental.pallas import tpu_sc as plsc

N = 10000
E = 320000
D_NODE = 128
D_EDGE = 16
G = 16
OUT = 128

BN = 2000
BEDGE = 8000


def _ln(h, g, b):
    mu = jnp.mean(h, axis=-1, keepdims=True)
    var = jnp.mean((h - mu) * (h - mu), axis=-1, keepdims=True)
    return (h - mu) * lax.rsqrt(var + 1e-5) * g + b


def _prep_body(x_ref, oh_ref, u_ref, wr_ref, ws_ref, wg_ref, b1_ref,
               ab_ref):
    xb = x_ref[...]
    ab_ref[0] = jnp.dot(xb, wr_ref[...], preferred_element_type=jnp.float32)
    ug = jnp.dot(u_ref[...], wg_ref[...], preferred_element_type=jnp.float32)
    ab_ref[1] = (jnp.dot(xb, ws_ref[...], preferred_element_type=jnp.float32)
                 + jnp.dot(oh_ref[...], ug, preferred_element_type=jnp.float32)
                 + b1_ref[...])


def _prep_tables(x, onehot, u, wr, ws, wg, b1):
    grid = N // BN
    full = lambda shape: pl.BlockSpec(shape, lambda i: (0, 0))
    return pl.pallas_call(
        _prep_body,
        grid=(grid,),
        in_specs=[
            pl.BlockSpec((BN, D_NODE), lambda i: (i, 0)),
            pl.BlockSpec((BN, G), lambda i: (i, 0)),
            full((G, D_NODE)),
            full((D_NODE, OUT)),
            full((D_NODE, OUT)),
            full((D_NODE, OUT)),
            full((1, OUT)),
        ],
        out_specs=pl.BlockSpec((2, BN, OUT), lambda i: (0, i, 0)),
        out_shape=jax.ShapeDtypeStruct((2, N, OUT), jnp.float32),
        compiler_params=pltpu.CompilerParams(
            dimension_semantics=("arbitrary",)),
    )(x, onehot, u, wr, ws, wg, b1)


def _edge_body(ga_ref, gb_ref, row_ref, st_ref, en_ref, e_ref, w1e_ref,
               w2_ref, b2_ref, g_ref, bt_ref, out_ref, ea_ref, acc_ref):
    i = pl.program_id(0)

    @pl.when(i == 0)
    def _():
        acc_ref[...] = jnp.zeros_like(acc_ref)

    h1 = jnp.maximum(
        ga_ref[...] + gb_ref[...]
        + jnp.dot(e_ref[...], w1e_ref[...],
                  preferred_element_type=jnp.float32), 0.0)
    h2 = jnp.maximum(
        jnp.dot(h1, w2_ref[...], preferred_element_type=jnp.float32)
        + b2_ref[...], 0.0)
    en = _ln(h2, g_ref[...], bt_ref[...])
    out_ref[...] = en
    rr = row_ref[0]
    ohgT = jnp.logical_and(rr >= st_ref[...], rr < en_ref[...]
                           ).astype(jnp.float32)
    acc_ref[...] += lax.dot_general(
        ohgT, en, (((1,), (0,)), ((), ())),
        preferred_element_type=jnp.float32)

    @pl.when(i == pl.num_programs(0) - 1)
    def _():
        ea_ref[...] = acc_ref[...]


def _edge_mlp(ga, gb, row3, starts, ends, e, w1e, w2, b2, g, bt,
              blk_off, n_blk):
    full = lambda shape: pl.BlockSpec(shape, lambda i: (0, 0))
    return pl.pallas_call(
        _edge_body,
        grid=(n_blk,),
        in_specs=[
            pl.BlockSpec((BEDGE, OUT), lambda i: (i, 0)),
            pl.BlockSpec((BEDGE, OUT), lambda i: (i, 0)),
            pl.BlockSpec((1, 1, BEDGE), lambda i: (i + blk_off, 0, 0)),
            pl.BlockSpec((G, 1), lambda i: (0, 0)),
            pl.BlockSpec((G, 1), lambda i: (0, 0)),
            pl.BlockSpec((BEDGE, D_EDGE), lambda i: (i + blk_off, 0)),
            full((D_EDGE, OUT)),
            full((OUT, OUT)),
            full((1, OUT)),
            full((1, OUT)),
            full((1, OUT)),
        ],
        out_specs=[
            pl.BlockSpec((BEDGE, OUT), lambda i: (i, 0)),
            pl.BlockSpec((G, OUT), lambda i: (0, 0)),
        ],
        out_shape=[
            jax.ShapeDtypeStruct((n_blk * BEDGE, OUT), jnp.float32),
            jax.ShapeDtypeStruct((G, OUT), jnp.float32),
        ],
        scratch_shapes=[pltpu.VMEM((G, OUT), jnp.float32)],
        compiler_params=pltpu.CompilerParams(
            dimension_semantics=("arbitrary",)),
    )(ga, gb, row3, starts, ends, e, w1e, w2, b2, g, bt)


def _node_body(x_ref, recv0_ref, recv1_ref, ea_ref, oh_ref, u_ref,
               nw1x_ref, nw1r_ref, nw1u_ref, nb1_ref, nw2_ref, nb2_ref,
               ng_ref, nbt_ref,
               gw1u_ref, gw1n_ref, gw1e_ref, gb1_ref, gw2_ref, gb2_ref,
               gg_ref, gbt_ref,
               xn_ref, un_ref, acc_ref):
    i = pl.program_id(0)

    @pl.when(i == 0)
    def _():
        acc_ref[...] = jnp.zeros_like(acc_ref)

    oh = oh_ref[...]
    un_tab = jnp.dot(u_ref[...], nw1u_ref[...],
                     preferred_element_type=jnp.float32)
    h = jnp.maximum(
        jnp.dot(x_ref[...], nw1x_ref[...], preferred_element_type=jnp.float32)
        + jnp.dot(recv0_ref[...] + recv1_ref[...], nw1r_ref[...],
                  preferred_element_type=jnp.float32)
        + jnp.dot(oh, un_tab, preferred_element_type=jnp.float32)
        + nb1_ref[...], 0.0)
    h2 = jnp.maximum(
        jnp.dot(h, nw2_ref[...], preferred_element_type=jnp.float32)
        + nb2_ref[...], 0.0)
    xn = _ln(h2, ng_ref[...], nbt_ref[...])
    xn_ref[...] = xn

    contract0 = (((0,), (0,)), ((), ()))
    acc_ref[...] += lax.dot_general(
        oh, xn, contract0, preferred_element_type=jnp.float32)

    @pl.when(i == pl.num_programs(0) - 1)
    def _():
        na = acc_ref[...]
        ea = ea_ref[...]
        hg = jnp.maximum(
            jnp.dot(u_ref[...], gw1u_ref[...],
                    preferred_element_type=jnp.float32)
            + jnp.dot(na, gw1n_ref[...], preferred_element_type=jnp.float32)
            + jnp.dot(ea, gw1e_ref[...], preferred_element_type=jnp.float32)
            + gb1_ref[...], 0.0)
        hg2 = jnp.maximum(
            jnp.dot(hg, gw2_ref[...], preferred_element_type=jnp.float32)
            + gb2_ref[...], 0.0)
        un_ref[...] = _ln(hg2, gg_ref[...], gbt_ref[...])


def _node_global(x, recv0, recv1, ea, onehot, u,
                 nw1x, nw1r, nw1u, nb1, nw2, nb2, ng, nbt,
                 gw1u, gw1n, gw1e, gb1, gw2, gb2, gg, gbt):
    grid = N // BN
    full = lambda shape: pl.BlockSpec(shape, lambda i: (0, 0))
    blk = lambda w: pl.BlockSpec((BN, w), lambda i: (i, 0))
    return pl.pallas_call(
        _node_body,
        grid=(grid,),
        in_specs=[
            blk(D_NODE), blk(OUT), blk(OUT),
            full((G, OUT)),
            blk(G),
            full((G, D_NODE)),
            full((D_NODE, OUT)), full((OUT, OUT)), full((D_NODE, OUT)),
            full((1, OUT)), full((OUT, OUT)), full((1, OUT)),
            full((1, OUT)), full((1, OUT)),
            full((D_NODE, OUT)), full((OUT, OUT)), full((OUT, OUT)),
            full((1, OUT)), full((OUT, OUT)), full((1, OUT)),
            full((1, OUT)), full((1, OUT)),
        ],
        out_specs=[
            pl.BlockSpec((BN, OUT), lambda i: (i, 0)),
            pl.BlockSpec((G, OUT), lambda i: (0, 0)),
        ],
        out_shape=[
            jax.ShapeDtypeStruct((N, OUT), jnp.float32),
            jax.ShapeDtypeStruct((G, OUT), jnp.float32),
        ],
        scratch_shapes=[pltpu.VMEM((G, OUT), jnp.float32)],
        compiler_params=pltpu.CompilerParams(
            dimension_semantics=("arbitrary",)),
    )(x, recv0, recv1, ea, onehot, u,
      nw1x, nw1r, nw1u, nb1, nw2, nb2, ng, nbt,
      gw1u, gw1n, gw1e, gb1, gw2, gb2, gg, gbt)


_SC_KB = 80
_SC_SUB = 8
_SC_CHUNK = _SC_KB * _SC_SUB

_NPAD = 10240
_HN = _NPAD // 2
_TRASH = _HN
_ACC = _HN + 8


def _sc_scatter2(e3_parts, ei4, zrows):
    info = plsc.get_sparse_core_info()
    ns = info.num_subcores
    part_rows = [p.shape[0] // _SC_SUB for p in e3_parts]
    stripe = _HN // ns
    mesh = plsc.VectorSubcoreMesh(core_axis_name="c", subcore_axis_name="s")

    @functools.partial(
        pl.kernel, mesh=mesh,
        out_type=jax.ShapeDtypeStruct((_NPAD, OUT), jnp.float32),
        scratch_types=[
            pltpu.VMEM((_SC_SUB, _SC_KB, OUT), jnp.float32),
            pltpu.VMEM((_SC_SUB, _SC_KB), jnp.int32),
            pltpu.VMEM_SHARED((_ACC, OUT), jnp.float32),
            pltpu.SemaphoreType.DMA,
        ],
    )
    def k(*refs):
        e_hbms = refs[:len(e3_parts)]
        ei_hbm, z_hbm, outr_hbm, data_v, idx_v, acc_sh, sem = \
            refs[len(e3_parts):]
        c = lax.axis_index("c")
        s = lax.axis_index("s")
        base = c * _HN

        pltpu.sync_copy(z_hbm.at[pl.ds(0, stripe)],
                        acc_sh.at[pl.ds(s * stripe, stripe)])
        plsc.subcore_barrier()

        def do_row(e_hbm, local, glob):
            cp1 = pltpu.make_async_copy(
                e_hbm.at[pl.ds(local * _SC_SUB, _SC_SUB)], data_v, sem)
            cp1.start()
            cp2 = pltpu.make_async_copy(ei_hbm.at[1, glob], idx_v, sem)
            cp2.start()
            cp1.wait()
            cp2.wait()
            for kk in range(_SC_SUB):
                for jj in range(_SC_KB // 16):
                    v = idx_v[kk, pl.ds(jj * 16, 16)] - base
                    oob = (v < 0) | (v >= _HN)
                    idx_v[kk, pl.ds(jj * 16, 16)] = jnp.where(oob, _TRASH, v)
            for kk in range(_SC_SUB):
                pltpu.sync_copy(data_v.at[kk], acc_sh.at[idx_v.at[kk]],
                                add=True)

        row_lo = 0
        for e_hbm, rows_p in zip(e_hbms, part_rows):
            n_rounds = rows_p // ns
            n_left = rows_p - n_rounds * ns

            def body(j, _, e_hbm=e_hbm, lo=row_lo):
                local = j * ns + s
                do_row(e_hbm, local, lo + local)
                return 0

            lax.fori_loop(0, n_rounds, body, 0)

            @pl.when(s < n_left)
            def _(e_hbm=e_hbm, lo=row_lo, n_rounds=n_rounds):
                local = n_rounds * ns + s
                do_row(e_hbm, local, lo + local)

            row_lo += rows_p

        plsc.subcore_barrier()
        pltpu.sync_copy(acc_sh.at[pl.ds(s * stripe, stripe)],
                        outr_hbm.at[pl.ds(base + s * stripe, stripe)])

    return k(*e3_parts, ei4, zrows)


def _sc_gather2(tab2, gi4, row_lo, row_n):
    info = plsc.get_sparse_core_info()
    ns = info.num_subcores
    n_rounds = row_n // ns
    n_left = row_n - n_rounds * ns
    mesh = plsc.VectorSubcoreMesh(core_axis_name="c", subcore_axis_name="s")

    @functools.partial(
        pl.kernel, mesh=mesh,
        out_type=jax.ShapeDtypeStruct((2, row_n * _SC_SUB, _SC_KB, OUT),
                                      jnp.float32),
        scratch_types=[
            pltpu.VMEM((_SC_SUB, _SC_KB, OUT), jnp.float32),
            pltpu.VMEM((_SC_SUB, _SC_KB), jnp.int32),
            pltpu.SemaphoreType.DMA,
        ],
    )
    def k(tab_hbm, gi_hbm, out_hbm, data_v, idx_v, sem):
        c = lax.axis_index("c")
        s = lax.axis_index("s")

        def do_row(local):
            pltpu.sync_copy(gi_hbm.at[c, row_lo + local], idx_v)
            for kk in range(_SC_SUB):
                pltpu.make_async_copy(
                    tab_hbm.at[idx_v.at[kk]], data_v.at[kk], sem).start()
            for kk in range(_SC_SUB):
                pltpu.make_async_copy(
                    tab_hbm.at[idx_v.at[kk]], data_v.at[kk], sem).wait()
            pltpu.sync_copy(
                data_v, out_hbm.at[c, pl.ds(local * _SC_SUB, _SC_SUB)])

        def body(j, _):
            do_row(j * ns + s)
            return 0

        lax.fori_loop(0, n_rounds, body, 0)

        @pl.when(s < n_left)
        def _():
            do_row(n_rounds * ns + s)

    return k(tab2, gi4)


def kernel(x, e, u, edge_index, batch,
           eW1, eb1, eW2, eb2, eg, ebt,
           nW1, nb1, nW2, nb2, ng, nbt,
           gW1, gb1, gW2, gb2, gg, gbt):
    row = edge_index[0]
    col = edge_index[1]
    onehot = (batch[:, None] == jnp.arange(G, dtype=jnp.int32)[None, :]
              ).astype(jnp.float32)

    r1 = lambda v: v.reshape(1, -1)

    w1e = eW1[:D_EDGE]
    w1r = eW1[D_EDGE:D_EDGE + D_NODE]
    w1s = eW1[D_EDGE + D_NODE:D_EDGE + 2 * D_NODE]
    w1g = eW1[D_EDGE + 2 * D_NODE:]

    ab_tab = _prep_tables(x, onehot, u, w1r, w1s, w1g, r1(eb1))
    tab2 = ab_tab.reshape(2 * N, OUT)
    gi4 = jnp.stack([col, row + N]).reshape(2, E // _SC_CHUNK,
                                            _SC_SUB, _SC_KB)
    cnt = jnp.sum(onehot, axis=0).astype(jnp.int32)
    ends_f = jnp.cumsum(cnt)
    starts = (ends_f - cnt).reshape(G, 1)
    ends = ends_f.reshape(G, 1)
    row3 = row.reshape(E // BEDGE, 1, BEDGE)

    K = 4
    n_rows_total = E // _SC_CHUNK
    part_rows = n_rows_total // K
    part_blk = (part_rows * _SC_CHUNK) // BEDGE
    ei4 = edge_index.reshape(2, E // _SC_CHUNK, _SC_SUB, _SC_KB)
    zrows = jnp.zeros((320, OUT), jnp.float32)

    e_parts, ea_parts, recvs = [], [], []
    for p in range(K):
        g2 = _sc_gather2(tab2, gi4, p * part_rows, part_rows)
        ga = g2[0].reshape(part_rows * _SC_CHUNK, OUT)
        gb = g2[1].reshape(part_rows * _SC_CHUNK, OUT)
        en_p, ea_p = _edge_mlp(ga, gb, row3, starts, ends, e, w1e, eW2,
                               r1(eb2), r1(eg), r1(ebt),
                               p * part_blk, part_blk)
        e_parts.append(en_p)
        ea_parts.append(ea_p)
        e3_p = en_p.reshape(part_rows * _SC_SUB, _SC_KB, OUT)
        recvs.append(_sc_scatter2([e3_p], ei4[:, p * part_rows:
                                               (p + 1) * part_rows],
                                  zrows))

    ea = sum(ea_parts[1:], ea_parts[0])
    recv_a = recvs[0] + recvs[1]
    recv_b = recvs[2] + recvs[3]
    e_new = jnp.concatenate(e_parts, axis=0)

    x_new, u_new = _node_global(
        x, recv_a, recv_b, ea, onehot, u,
        nW1[:D_NODE], nW1[D_NODE:D_NODE + OUT], nW1[D_NODE + OUT:],
        r1(nb1), nW2, r1(nb2), r1(ng), r1(nbt),
        gW1[:D_NODE], gW1[D_NODE:D_NODE + OUT], gW1[D_NODE + OUT:],
        r1(gb1), gW2, r1(gb2), r1(gg), r1(gbt))

    return (x_new, e_new, u_new)
